# baseline (speedup 1.0000x reference)
import functools

import jax
import jax.numpy as jnp
from jax import lax
from jax.experimental import pallas as pl
from jax.experimental.pallas import tpu as pltpu
from jax.experimental.pallas import tpu_sc as plsc

N = 10000
E = 320000
D = 128
H = 128
C = 40

NC = 2
NS = 16
NW = NC * NS
CHUNK = 64
NCH = 160
EPW = NCH * CHUNK
EPAD = NW * EPW
TRASH = N + 200
NPAD = 10240
RPT = NPAD // NS
ND = 4
NI = 8

_mesh = plsc.VectorSubcoreMesh(core_axis_name="c", subcore_axis_name="s")


@functools.partial(
    pl.kernel, mesh=_mesh,
    out_type=jax.ShapeDtypeStruct((NC, NPAD, D), jnp.float32),
    scratch_types=(
        [pltpu.VMEM((CHUNK,), jnp.int32) for _ in range(NI)]
        + [pltpu.VMEM((CHUNK, D), jnp.float32), pltpu.VMEM((CHUNK, D),
                                                           jnp.float32)]
        + [pltpu.VMEM_SHARED((NPAD, D), jnp.float32)]
        + [pltpu.SemaphoreType.DMA for _ in range(NI + ND)]
    ),
)
def _deg_sc(col_hbm, out_hbm, *refs):
    cbufs = refs[:NI]
    zbuf, ones = refs[NI], refs[NI + 1]
    acc = refs[NI + 2]
    csems = refs[NI + 3:NI + 3 + NI]
    ssems = refs[NI + 3 + NI:]
    c = lax.axis_index("c")
    s = lax.axis_index("s")
    w = s * NC + c

    def _fill(ref, v):
        def _row(i, _):
            for j in range(D // 16):
                ref[i, pl.ds(j * 16, 16)] = jnp.full((16,), v, jnp.float32)
            return 0
        lax.fori_loop(0, CHUNK, _row, 0)

    _fill(zbuf, 0.0)
    _fill(ones, 1.0)

    def _zacc(j, _):
        pltpu.sync_copy(zbuf, acc.at[pl.ds(s * RPT + j * CHUNK, CHUNK)])
        return 0
    lax.fori_loop(0, RPT // CHUNK, _zacc, 0)
    plsc.subcore_barrier()

    def _cst(k, m):
        pltpu.make_async_copy(col_hbm.at[pl.ds(w * EPW + k * CHUNK, CHUNK)],
                              cbufs[m], csems[m]).start()

    def _cwt(m):
        pltpu.make_async_copy(col_hbm.at[pl.ds(0, CHUNK)], cbufs[m],
                              csems[m]).wait()

    def _sst(m, sd):
        pltpu.make_async_copy(ones, acc.at[cbufs[m]],
                              ssems[sd]).start(add=True)

    def _swt(m, sd):
        pltpu.make_async_copy(ones, acc.at[cbufs[m]], ssems[sd]).wait()

    for k in range(NI):
        _cst(k, k)
    for k in (0, 1):
        _cwt(k)
        _sst(k, k)

    def _steady(i, _):
        k0 = 8 * i + 2
        for j in range(8):
            _swt(j % 4, j % 4)
            _cst(k0 + j + 6, j)
            _cwt((2 + j) % 8)
            _sst((2 + j) % 8, (2 + j) % 4)
        return 0
    lax.fori_loop(0, 19, _steady, 0)

    for k in range(154, 160):
        _swt((k - 2) % 8, (k - 2) % 4)
        _cwt(k % 8)
        _sst(k % 8, k % 4)
    _swt(158 % 8, 158 % 4)
    _swt(159 % 8, 159 % 4)
    plsc.subcore_barrier()

    pltpu.sync_copy(acc.at[pl.ds(s * RPT, RPT)],
                    out_hbm.at[c, pl.ds(s * RPT, RPT)])


@functools.partial(
    pl.kernel, mesh=_mesh,
    out_type=jax.ShapeDtypeStruct((NC, NPAD, D), jnp.float32),
    scratch_types=(
        [pltpu.VMEM((CHUNK,), jnp.int32) for _ in range(2 * NI)]
        + [pltpu.VMEM((CHUNK, D), jnp.float32) for _ in range(ND)]
        + [pltpu.VMEM_SHARED((NPAD, D), jnp.float32)]
        + [pltpu.SemaphoreType.DMA for _ in range(2 * NI + 2 * ND)]
    ),
)
def _prop_sc(row_hbm, col_hbm, g_hbm, out_hbm, *refs):
    rbufs = refs[:NI]
    cbufs = refs[NI:2 * NI]
    bufs = refs[2 * NI:2 * NI + ND]
    acc = refs[2 * NI + ND]
    sems = refs[2 * NI + ND + 1:]
    rsems = sems[:NI]
    csems = sems[NI:2 * NI]
    gsems = sems[2 * NI:2 * NI + ND]
    ssems = sems[2 * NI + ND:]
    c = lax.axis_index("c")
    s = lax.axis_index("s")
    w = s * NC + c

    def _zrow(i, _):
        for j in range(D // 16):
            bufs[0][i, pl.ds(j * 16, 16)] = jnp.zeros((16,), jnp.float32)
        return 0
    lax.fori_loop(0, CHUNK, _zrow, 0)

    def _zacc(j, _):
        pltpu.sync_copy(bufs[0], acc.at[pl.ds(s * RPT + j * CHUNK, CHUNK)])
        return 0
    lax.fori_loop(0, RPT // CHUNK, _zacc, 0)
    plsc.subcore_barrier()

    def _ist(k, m):
        base = w * EPW + k * CHUNK
        pltpu.make_async_copy(row_hbm.at[pl.ds(base, CHUNK)], rbufs[m],
                              rsems[m]).start()
        pltpu.make_async_copy(col_hbm.at[pl.ds(base, CHUNK)], cbufs[m],
                              csems[m]).start()

    def _rwt(m):
        pltpu.make_async_copy(row_hbm.at[pl.ds(0, CHUNK)], rbufs[m],
                              rsems[m]).wait()

    def _cwt(m):
        pltpu.make_async_copy(col_hbm.at[pl.ds(0, CHUNK)], cbufs[m],
                              csems[m]).wait()

    def _gst(m, sd):
        pltpu.make_async_copy(g_hbm.at[rbufs[m]], bufs[sd], gsems[sd]).start()

    def _gwt(m, sd):
        pltpu.make_async_copy(g_hbm.at[rbufs[m]], bufs[sd], gsems[sd]).wait()

    def _sst(m, sd):
        pltpu.make_async_copy(bufs[sd], acc.at[cbufs[m]],
                              ssems[sd]).start(add=True)

    def _swt(m, sd):
        pltpu.make_async_copy(bufs[sd], acc.at[cbufs[m]], ssems[sd]).wait()

    for k in range(NI):
        _ist(k, k)
    for k in (0, 1):
        _rwt(k)
        _gst(k, k)
    for k in (0, 1):
        _rwt(k + 2)
        _gst(k + 2, k + 2)
        _gwt(k, k)
        _cwt(k)
        _sst(k, k)

    def _steady(i, _):
        k0 = 8 * i + 2
        for j in range(8):
            _swt(j % 8, j % 4)
            _ist(k0 + j + 6, j)
            _rwt((4 + j) % 8)
            _gst((4 + j) % 8, j % 4)
            _gwt((2 + j) % 8, (2 + j) % 4)
            _cwt((2 + j) % 8)
            _sst((2 + j) % 8, (2 + j) % 4)
        return 0
    lax.fori_loop(0, 19, _steady, 0)

    for k in range(154, 160):
        _swt((k - 2) % 8, (k - 2) % 4)
        if k + 2 < NCH:
            _rwt((k + 2) % 8)
            _gst((k + 2) % 8, (k + 2) % 4)
        _gwt(k % 8, k % 4)
        _cwt(k % 8)
        _sst(k % 8, k % 4)
    _swt(158 % 8, 158 % 4)
    _swt(159 % 8, 159 % 4)
    plsc.subcore_barrier()

    pltpu.sync_copy(acc.at[pl.ds(s * RPT, RPT)],
                    out_hbm.at[c, pl.ds(s * RPT, RPT)])


_BR = 2000


def _init_tc(d0, d1, x, g0, dis, dis2):
    deg = d0[...] + d1[...] + 1.0
    r = lax.rsqrt(deg)
    dis[...] = r
    dis2[...] = 1.0 / deg
    g0[...] = x[...] * r


def _upd_tc(p0, p1, g, dis2, out):
    out[...] = (p0[0] + p1[0] + g[...]) * dis2[...]


def _mlp_tc(p0, p1, g, dis, wg, bg, w1, b1, w2, b2, wo, out):
    h = (p0[0] + p1[0] + g[...]) * dis[...]
    h = jnp.dot(h, wg[...], preferred_element_type=jnp.float32) + bg[...]
    h = jnp.maximum(jnp.dot(h, w1[...], preferred_element_type=jnp.float32)
                    + b1[...], 0.0)
    h = jnp.maximum(jnp.dot(h, w2[...], preferred_element_type=jnp.float32)
                    + b2[...], 0.0)
    logits = jnp.dot(h, wo[...], preferred_element_type=jnp.float32)
    valid = lax.broadcasted_iota(jnp.int32, logits.shape, 1) < C
    logits = jnp.where(valid, logits, -1e30)
    m = jnp.max(logits, axis=1, keepdims=True)
    lse = jnp.log(jnp.sum(jnp.exp(logits - m), axis=1, keepdims=True)) + m
    out[...] = logits - lse


def _row_spec(bc):
    return pl.BlockSpec((_BR, bc), lambda i: (i, 0))


def _part_spec(core):
    return pl.BlockSpec((1, _BR, D), lambda i, _c=core: (_c, i, 0))


def _full_spec(r, c):
    return pl.BlockSpec((r, c), lambda i: (0, 0))


_init_call = pl.pallas_call(
    _init_tc,
    grid=(N // _BR,),
    in_specs=[_row_spec(1), _row_spec(1), _row_spec(D)],
    out_specs=[_row_spec(D), _row_spec(1), _row_spec(1)],
    out_shape=[
        jax.ShapeDtypeStruct((N, D), jnp.float32),
        jax.ShapeDtypeStruct((N, 1), jnp.float32),
        jax.ShapeDtypeStruct((N, 1), jnp.float32),
    ],
)

_upd_call = pl.pallas_call(
    _upd_tc,
    grid=(N // _BR,),
    in_specs=[_part_spec(0), _part_spec(1), _row_spec(D), _row_spec(1)],
    out_specs=_row_spec(D),
    out_shape=jax.ShapeDtypeStruct((N, D), jnp.float32),
)

_mlp_call = pl.pallas_call(
    _mlp_tc,
    grid=(N // _BR,),
    in_specs=[_part_spec(0), _part_spec(1), _row_spec(D), _row_spec(1),
              _full_spec(D, H), _full_spec(1, H),
              _full_spec(H, H), _full_spec(1, H),
              _full_spec(H, H), _full_spec(1, H),
              _full_spec(H, 128)],
    out_specs=_row_spec(128),
    out_shape=jax.ShapeDtypeStruct((N, 128), jnp.float32),
)


def kernel(x, edge_index, W_gcn, b_gcn, W_h1, b_h1, W_h2, b_h2, W_out):
    row = edge_index[0]
    col = edge_index[1]
    npad_e = EPAD - E
    rowp = jnp.concatenate([row, jnp.zeros((npad_e,), jnp.int32)])
    colp = jnp.concatenate([col, jnp.full((npad_e,), TRASH, jnp.int32)])

    degp = _deg_sc(colp)
    d0 = degp[0, :N, 0:1]
    d1 = degp[1, :N, 0:1]
    g, dis, dis2 = _init_call(d0, d1, x)

    for _ in range(3):
        p = _prop_sc(rowp, colp, g)
        g = _upd_call(p, p, g, dis2)
    p = _prop_sc(rowp, colp, g)

    wo_pad = jnp.pad(W_out, ((0, 0), (0, 128 - C)))
    out = _mlp_call(p, p, g, dis,
                    W_gcn, b_gcn.reshape(1, H),
                    W_h1, b_h1.reshape(1, H),
                    W_h2, b_h2.reshape(1, H),
                    wo_pad)
    return out[:, :C]


# speedup vs baseline: 2.3898x; 2.3898x over previous
import functools

import jax
import jax.numpy as jnp
from jax import lax
from jax.experimental import pallas as pl
from jax.experimental.pallas import tpu as pltpu
from jax.experimental.pallas import tpu_sc as plsc

N = 10000
E = 320000
D = 128
H = 128
C = 40

NC = 2
NS = 16
NW = NC * NS
EPW = E // NW
CHUNK = 80
NCH = EPW // CHUNK
NPAD = 10240
RPT = NPAD // NS

_mesh = plsc.VectorSubcoreMesh(core_axis_name="c", subcore_axis_name="s")


@functools.partial(
    pl.kernel, mesh=_mesh,
    out_type=jax.ShapeDtypeStruct((NC, NPAD, D), jnp.float32),
    scratch_types=[
        pltpu.VMEM((CHUNK,), jnp.int32),
        pltpu.VMEM((CHUNK,), jnp.int32),
        pltpu.VMEM((CHUNK, D), jnp.float32),
        pltpu.VMEM((CHUNK, D), jnp.float32),
        pltpu.VMEM_SHARED((NPAD, D), jnp.float32),
        pltpu.SemaphoreType.DMA,
        pltpu.SemaphoreType.DMA,
        pltpu.SemaphoreType.DMA,
        pltpu.SemaphoreType.DMA,
    ],
)
def _deg_sc(col_hbm, out_hbm, cb0, cb1, zbuf, ones, acc, c0, c1, s0, s1):
    c = lax.axis_index("c")
    s = lax.axis_index("s")
    w = s * NC + c
    cbufs = (cb0, cb1)
    csems = (c0, c1)
    ssems = (s0, s1)

    def _fill(ref, v):
        def _row(i, _):
            for j in range(D // 16):
                ref[i, pl.ds(j * 16, 16)] = jnp.full((16,), v, jnp.float32)
            return 0
        lax.fori_loop(0, CHUNK, _row, 0)

    _fill(zbuf, 0.0)
    _fill(ones, 1.0)

    def _zacc(j, _):
        pltpu.sync_copy(zbuf, acc.at[pl.ds(s * RPT + j * CHUNK, CHUNK)])
        return 0
    lax.fori_loop(0, RPT // CHUNK, _zacc, 0)
    plsc.subcore_barrier()

    def _cdesc(k, b):
        base = w * EPW + k * CHUNK
        return pltpu.make_async_copy(col_hbm.at[pl.ds(base, CHUNK)],
                                     cbufs[b], csems[b])

    def _sca(b):
        return pltpu.make_async_copy(ones, acc.at[cbufs[b]], ssems[b])

    _cdesc(0, 0).start()

    def _pair(i, _):
        k = 2 * i
        _cdesc(k, 0).wait()

        @pl.when(i > 0)
        def _():
            _sca(1).wait()
        _cdesc(k + 1, 1).start()
        _sca(0).start(add=True)

        _cdesc(k + 1, 1).wait()

        @pl.when(i < NCH // 2 - 1)
        def _():
            _sca(0).wait()
            _cdesc(k + 2, 0).start()
        _sca(1).start(add=True)
        return 0
    lax.fori_loop(0, NCH // 2, _pair, 0)

    kl = NCH - 1
    _sca(0).wait()
    _cdesc(kl, 0).start()
    _cdesc(kl, 0).wait()
    _sca(0).start(add=True)
    _sca(0).wait()
    _sca(1).wait()
    plsc.subcore_barrier()

    pltpu.sync_copy(acc.at[pl.ds(s * RPT, RPT)],
                    out_hbm.at[c, pl.ds(s * RPT, RPT)])


@functools.partial(
    pl.kernel, mesh=_mesh,
    out_type=jax.ShapeDtypeStruct((NC, NPAD, D), jnp.float32),
    scratch_types=[
        pltpu.VMEM((NCH, CHUNK), jnp.int32),
        pltpu.VMEM((CHUNK,), jnp.int32),
        pltpu.VMEM((CHUNK,), jnp.int32),
        pltpu.VMEM((CHUNK, D), jnp.float32),
        pltpu.VMEM((CHUNK, D), jnp.float32),
        pltpu.VMEM_SHARED((NPAD, D), jnp.float32),
        pltpu.SemaphoreType.DMA,
        pltpu.SemaphoreType.DMA,
        pltpu.SemaphoreType.DMA,
        pltpu.SemaphoreType.DMA,
        pltpu.SemaphoreType.DMA,
        pltpu.SemaphoreType.DMA,
    ],
)
def _prop_sc(row_hbm, col_hbm, g_hbm, out_hbm,
             idxrow, cb0, cb1, buf0, buf1, acc, g0, g1, c0, c1, s0, s1):
    c = lax.axis_index("c")
    s = lax.axis_index("s")
    w = s * NC + c
    bufs = (buf0, buf1)
    cbufs = (cb0, cb1)
    gsems = (g0, g1)
    csems = (c0, c1)
    ssems = (s0, s1)

    pltpu.sync_copy(row_hbm.at[w], idxrow)

    def _zrow(i, _):
        for j in range(D // 16):
            buf0[i, pl.ds(j * 16, 16)] = jnp.zeros((16,), jnp.float32)
        return 0
    lax.fori_loop(0, CHUNK, _zrow, 0)

    def _zacc(j, _):
        pltpu.sync_copy(buf0, acc.at[pl.ds(s * RPT + j * CHUNK, CHUNK)])
        return 0
    lax.fori_loop(0, RPT // CHUNK, _zacc, 0)
    plsc.subcore_barrier()

    def _cdesc(k, b):
        base = w * EPW + k * CHUNK
        return pltpu.make_async_copy(col_hbm.at[pl.ds(base, CHUNK)],
                                     cbufs[b], csems[b])

    def _start(k, b):
        _cdesc(k, b).start()
        pltpu.make_async_copy(g_hbm.at[idxrow.at[k]], bufs[b],
                              gsems[b]).start()

    def _wait_in(k, b):
        _cdesc(k, b).wait()
        pltpu.make_async_copy(g_hbm.at[idxrow.at[k]], bufs[b],
                              gsems[b]).wait()

    def _sca(b):
        return pltpu.make_async_copy(bufs[b], acc.at[cbufs[b]], ssems[b])

    _start(0, 0)

    def _pair(i, _):
        k = 2 * i
        _wait_in(k, 0)

        @pl.when(i > 0)
        def _():
            _sca(1).wait()
        _start(k + 1, 1)
        _sca(0).start(add=True)

        _wait_in(k + 1, 1)

        @pl.when(i < NCH // 2 - 1)
        def _():
            _sca(0).wait()
            _start(k + 2, 0)
        _sca(1).start(add=True)
        return 0
    lax.fori_loop(0, NCH // 2, _pair, 0)

    kl = NCH - 1
    _sca(0).wait()
    _start(kl, 0)
    _wait_in(kl, 0)
    _sca(0).start(add=True)
    _sca(0).wait()
    _sca(1).wait()
    plsc.subcore_barrier()

    pltpu.sync_copy(acc.at[pl.ds(s * RPT, RPT)],
                    out_hbm.at[c, pl.ds(s * RPT, RPT)])


_BR = 2000


def _init_tc(d0, d1, x, g0, dis, dis2):
    deg = d0[...] + d1[...] + 1.0
    r = lax.rsqrt(deg)
    dis[...] = r
    dis2[...] = 1.0 / deg
    g0[...] = x[...] * r


def _upd_tc(p0, p1, g, dis2, out):
    out[...] = (p0[0] + p1[0] + g[...]) * dis2[...]


def _mlp_tc(p0, p1, g, dis, wg, bg, w1, b1, w2, b2, wo, out):
    h = (p0[0] + p1[0] + g[...]) * dis[...]
    h = jnp.dot(h, wg[...], preferred_element_type=jnp.float32) + bg[...]
    h = jnp.maximum(jnp.dot(h, w1[...], preferred_element_type=jnp.float32)
                    + b1[...], 0.0)
    h = jnp.maximum(jnp.dot(h, w2[...], preferred_element_type=jnp.float32)
                    + b2[...], 0.0)
    logits = jnp.dot(h, wo[...], preferred_element_type=jnp.float32)
    valid = lax.broadcasted_iota(jnp.int32, logits.shape, 1) < C
    logits = jnp.where(valid, logits, -1e30)
    m = jnp.max(logits, axis=1, keepdims=True)
    lse = jnp.log(jnp.sum(jnp.exp(logits - m), axis=1, keepdims=True)) + m
    out[...] = logits - lse


def _row_spec(bc):
    return pl.BlockSpec((_BR, bc), lambda i: (i, 0))


def _part_spec(core):
    return pl.BlockSpec((1, _BR, D), lambda i, _c=core: (_c, i, 0))


def _full_spec(r, c):
    return pl.BlockSpec((r, c), lambda i: (0, 0))


_init_call = pl.pallas_call(
    _init_tc,
    grid=(N // _BR,),
    in_specs=[_row_spec(1), _row_spec(1), _row_spec(D)],
    out_specs=[_row_spec(D), _row_spec(1), _row_spec(1)],
    out_shape=[
        jax.ShapeDtypeStruct((N, D), jnp.float32),
        jax.ShapeDtypeStruct((N, 1), jnp.float32),
        jax.ShapeDtypeStruct((N, 1), jnp.float32),
    ],
)

_upd_call = pl.pallas_call(
    _upd_tc,
    grid=(N // _BR,),
    in_specs=[_part_spec(0), _part_spec(1), _row_spec(D), _row_spec(1)],
    out_specs=_row_spec(D),
    out_shape=jax.ShapeDtypeStruct((N, D), jnp.float32),
)

_mlp_call = pl.pallas_call(
    _mlp_tc,
    grid=(N // _BR,),
    in_specs=[_part_spec(0), _part_spec(1), _row_spec(D), _row_spec(1),
              _full_spec(D, H), _full_spec(1, H),
              _full_spec(H, H), _full_spec(1, H),
              _full_spec(H, H), _full_spec(1, H),
              _full_spec(H, 128)],
    out_specs=_row_spec(128),
    out_shape=jax.ShapeDtypeStruct((N, 128), jnp.float32),
)


def kernel(x, edge_index, W_gcn, b_gcn, W_h1, b_h1, W_h2, b_h2, W_out):
    row = edge_index[0]
    col = edge_index[1]
    row3 = row.reshape(NW, NCH, CHUNK)

    degp = _deg_sc(col)
    d0 = degp[0, :N, 0:1]
    d1 = degp[1, :N, 0:1]
    g, dis, dis2 = _init_call(d0, d1, x)

    for _ in range(3):
        p = _prop_sc(row3, col, g)
        g = _upd_call(p, p, g, dis2)
    p = _prop_sc(row3, col, g)

    wo_pad = jnp.pad(W_out, ((0, 0), (0, 128 - C)))
    out = _mlp_call(p, p, g, dis,
                    W_gcn, b_gcn.reshape(1, H),
                    W_h1, b_h1.reshape(1, H),
                    W_h2, b_h2.reshape(1, H),
                    wo_pad)
    return out[:, :C]


# speedup vs baseline: 2.9148x; 1.2197x over previous
import functools

import jax
import jax.numpy as jnp
from jax import lax
from jax.experimental import pallas as pl
from jax.experimental.pallas import tpu as pltpu
from jax.experimental.pallas import tpu_sc as plsc

N = 10000
E = 320000
D = 128
H = 128
C = 40

NC = 2
NS = 16
NW = NC * NS
EPW = E // NW
CHUNK = 80
NCH = EPW // CHUNK
NPAD = 10240
RPT = NPAD // NS

_mesh = plsc.VectorSubcoreMesh(core_axis_name="c", subcore_axis_name="s")


@functools.partial(
    pl.kernel, mesh=_mesh,
    out_type=jax.ShapeDtypeStruct((NC, NPAD, D), jnp.float32),
    scratch_types=[
        pltpu.VMEM((CHUNK,), jnp.int32),
        pltpu.VMEM((CHUNK,), jnp.int32),
        pltpu.VMEM((CHUNK, D), jnp.float32),
        pltpu.VMEM((CHUNK, D), jnp.float32),
        pltpu.VMEM_SHARED((NPAD, D), jnp.float32),
        pltpu.SemaphoreType.DMA,
        pltpu.SemaphoreType.DMA,
        pltpu.SemaphoreType.DMA,
        pltpu.SemaphoreType.DMA,
    ],
)
def _deg_sc(col_hbm, out_hbm, cb0, cb1, zbuf, ones, acc, c0, c1, s0, s1):
    c = lax.axis_index("c")
    s = lax.axis_index("s")
    w = s * NC + c
    cbufs = (cb0, cb1)
    csems = (c0, c1)
    ssems = (s0, s1)

    def _fill(ref, v):
        def _row(i, _):
            for j in range(D // 16):
                ref[i, pl.ds(j * 16, 16)] = jnp.full((16,), v, jnp.float32)
            return 0
        lax.fori_loop(0, CHUNK, _row, 0)

    _fill(zbuf, 0.0)
    _fill(ones, 1.0)

    def _zacc(j, _):
        pltpu.sync_copy(zbuf, acc.at[pl.ds(s * RPT + j * CHUNK, CHUNK)])
        return 0
    lax.fori_loop(0, RPT // CHUNK, _zacc, 0)
    plsc.subcore_barrier()

    def _cdesc(k, b):
        base = w * EPW + k * CHUNK
        return pltpu.make_async_copy(col_hbm.at[pl.ds(base, CHUNK)],
                                     cbufs[b], csems[b])

    def _sca(b):
        return pltpu.make_async_copy(ones, acc.at[cbufs[b]], ssems[b])

    _cdesc(0, 0).start()

    def _pair(i, _):
        k = 2 * i
        _cdesc(k, 0).wait()

        @pl.when(i > 0)
        def _():
            _sca(1).wait()
        _cdesc(k + 1, 1).start()
        _sca(0).start(add=True)

        _cdesc(k + 1, 1).wait()

        @pl.when(i < NCH // 2 - 1)
        def _():
            _sca(0).wait()
            _cdesc(k + 2, 0).start()
        _sca(1).start(add=True)
        return 0
    lax.fori_loop(0, NCH // 2, _pair, 0)

    kl = NCH - 1
    _sca(0).wait()
    _cdesc(kl, 0).start()
    _cdesc(kl, 0).wait()
    _sca(0).start(add=True)
    _sca(0).wait()
    _sca(1).wait()
    plsc.subcore_barrier()

    pltpu.sync_copy(acc.at[pl.ds(s * RPT, RPT)],
                    out_hbm.at[c, pl.ds(s * RPT, RPT)])


@functools.partial(
    pl.kernel, mesh=_mesh,
    out_type=jax.ShapeDtypeStruct((NC, NPAD, D), jnp.float32),
    scratch_types=[
        pltpu.VMEM((NCH, CHUNK), jnp.int32),
        pltpu.VMEM((CHUNK,), jnp.int32),
        pltpu.VMEM((CHUNK,), jnp.int32),
        pltpu.VMEM((CHUNK, D), jnp.float32),
        pltpu.VMEM((CHUNK, D), jnp.float32),
        pltpu.VMEM_SHARED((NPAD, D), jnp.float32),
        pltpu.SemaphoreType.DMA,
        pltpu.SemaphoreType.DMA,
        pltpu.SemaphoreType.DMA,
        pltpu.SemaphoreType.DMA,
        pltpu.SemaphoreType.DMA,
        pltpu.SemaphoreType.DMA,
    ],
)
def _prop_sc(row_hbm, col_hbm, g_hbm, out_hbm,
             idxrow, cb0, cb1, buf0, buf1, acc, g0, g1, c0, c1, s0, s1):
    c = lax.axis_index("c")
    s = lax.axis_index("s")
    w = s * NC + c
    bufs = (buf0, buf1)
    cbufs = (cb0, cb1)
    gsems = (g0, g1)
    csems = (c0, c1)
    ssems = (s0, s1)

    pltpu.sync_copy(row_hbm.at[w], idxrow)

    def _zrow(i, _):
        for j in range(D // 16):
            buf0[i, pl.ds(j * 16, 16)] = jnp.zeros((16,), jnp.float32)
        return 0
    lax.fori_loop(0, CHUNK, _zrow, 0)

    def _zacc(j, _):
        pltpu.sync_copy(buf0, acc.at[pl.ds(s * RPT + j * CHUNK, CHUNK)])
        return 0
    lax.fori_loop(0, RPT // CHUNK, _zacc, 0)
    plsc.subcore_barrier()

    def _cdesc(k, b):
        base = w * EPW + k * CHUNK
        return pltpu.make_async_copy(col_hbm.at[pl.ds(base, CHUNK)],
                                     cbufs[b], csems[b])

    def _start(k, b):
        _cdesc(k, b).start()
        pltpu.make_async_copy(g_hbm.at[idxrow.at[k]], bufs[b],
                              gsems[b]).start()

    def _wait_in(k, b):
        _cdesc(k, b).wait()
        pltpu.make_async_copy(g_hbm.at[idxrow.at[k]], bufs[b],
                              gsems[b]).wait()

    def _sca(b):
        return pltpu.make_async_copy(bufs[b], acc.at[cbufs[b]], ssems[b])

    _start(0, 0)

    def _pair(i, _):
        k = 2 * i

        @pl.when(i > 0)
        def _():
            _sca(1).wait()
        _start(k + 1, 1)
        _wait_in(k, 0)
        _sca(0).start(add=True)

        _wait_in(k + 1, 1)

        @pl.when(i < NCH // 2 - 1)
        def _():
            _sca(0).wait()
            _start(k + 2, 0)
        _sca(1).start(add=True)
        return 0
    lax.fori_loop(0, NCH // 2, _pair, 0)

    kl = NCH - 1
    _sca(0).wait()
    _start(kl, 0)
    _wait_in(kl, 0)
    _sca(0).start(add=True)
    _sca(0).wait()
    _sca(1).wait()
    plsc.subcore_barrier()

    pltpu.sync_copy(acc.at[pl.ds(s * RPT, RPT)],
                    out_hbm.at[c, pl.ds(s * RPT, RPT)])


_BR = 2000


def _init_tc(d0, d1, x, g0, dis, dis2):
    deg = d0[...] + d1[...] + 1.0
    r = lax.rsqrt(deg)
    dis[...] = r
    dis2[...] = 1.0 / deg
    g0[...] = x[...] * r


def _upd_tc(p0, p1, g, dis2, out):
    out[...] = (p0[0] + p1[0] + g[...]) * dis2[...]


def _mlp_tc(p0, p1, g, dis, wg, bg, w1, b1, w2, b2, wo, out):
    h = (p0[0] + p1[0] + g[...]) * dis[...]
    h = jnp.dot(h, wg[...], preferred_element_type=jnp.float32) + bg[...]
    h = jnp.maximum(jnp.dot(h, w1[...], preferred_element_type=jnp.float32)
                    + b1[...], 0.0)
    h = jnp.maximum(jnp.dot(h, w2[...], preferred_element_type=jnp.float32)
                    + b2[...], 0.0)
    logits = jnp.dot(h, wo[...], preferred_element_type=jnp.float32)
    valid = lax.broadcasted_iota(jnp.int32, logits.shape, 1) < C
    logits = jnp.where(valid, logits, -1e30)
    m = jnp.max(logits, axis=1, keepdims=True)
    lse = jnp.log(jnp.sum(jnp.exp(logits - m), axis=1, keepdims=True)) + m
    out[...] = logits - lse


def _row_spec(bc):
    return pl.BlockSpec((_BR, bc), lambda i: (i, 0))


def _part_spec(core):
    return pl.BlockSpec((1, _BR, D), lambda i, _c=core: (_c, i, 0))


def _full_spec(r, c):
    return pl.BlockSpec((r, c), lambda i: (0, 0))


_init_call = pl.pallas_call(
    _init_tc,
    grid=(N // _BR,),
    in_specs=[_row_spec(1), _row_spec(1), _row_spec(D)],
    out_specs=[_row_spec(D), _row_spec(1), _row_spec(1)],
    out_shape=[
        jax.ShapeDtypeStruct((N, D), jnp.float32),
        jax.ShapeDtypeStruct((N, 1), jnp.float32),
        jax.ShapeDtypeStruct((N, 1), jnp.float32),
    ],
)

_upd_call = pl.pallas_call(
    _upd_tc,
    grid=(N // _BR,),
    in_specs=[_part_spec(0), _part_spec(1), _row_spec(D), _row_spec(1)],
    out_specs=_row_spec(D),
    out_shape=jax.ShapeDtypeStruct((N, D), jnp.float32),
)

_mlp_call = pl.pallas_call(
    _mlp_tc,
    grid=(N // _BR,),
    in_specs=[_part_spec(0), _part_spec(1), _row_spec(D), _row_spec(1),
              _full_spec(D, H), _full_spec(1, H),
              _full_spec(H, H), _full_spec(1, H),
              _full_spec(H, H), _full_spec(1, H),
              _full_spec(H, 128)],
    out_specs=_row_spec(128),
    out_shape=jax.ShapeDtypeStruct((N, 128), jnp.float32),
)


def kernel(x, edge_index, W_gcn, b_gcn, W_h1, b_h1, W_h2, b_h2, W_out):
    row = edge_index[0]
    col = edge_index[1]
    row3 = row.reshape(NW, NCH, CHUNK)

    degp = _deg_sc(col)
    d0 = degp[0, :N, 0:1]
    d1 = degp[1, :N, 0:1]
    g, dis, dis2 = _init_call(d0, d1, x)

    for _ in range(3):
        p = _prop_sc(row3, col, g)
        g = _upd_call(p, p, g, dis2)
    p = _prop_sc(row3, col, g)

    wo_pad = jnp.pad(W_out, ((0, 0), (0, 128 - C)))
    out = _mlp_call(p, p, g, dis,
                    W_gcn, b_gcn.reshape(1, H),
                    W_h1, b_h1.reshape(1, H),
                    W_h2, b_h2.reshape(1, H),
                    wo_pad)
    return out[:, :C]
